# Initial kernel scaffold; baseline (speedup 1.0000x reference)
import jax
import jax.numpy as jnp
from jax import lax
from jax.experimental import pallas as pl
from jax.experimental.pallas import tpu as pltpu
from jax.experimental.pallas import tpu_sc as plsc

B = 16384
EMB = 32
N_GENRES = 4
TEXT_LEN = 20
GENRE_VOCAB = 21

NUM_WORKERS = 32
ROWS_PER_WORKER = B // NUM_WORKERS
CHUNK = 64
NCHUNKS = ROWS_PER_WORKER // CHUNK
TOK_PER_CHUNK = CHUNK * TEXT_LEN
IDX_W = 128
NGATHER = TOK_PER_CHUNK // IDX_W


def _body(title_idx, genres_t, text_idx2, text_flat,
          title_tab, genre_tab, text_tab, out,
          gtab_v, tidx_v, trows_v, gidx_v, xidx_v, xtok_v, xrows_v,
          out_v, sem):
    wid = lax.axis_index("s") * 2 + lax.axis_index("c")
    base = wid * ROWS_PER_WORKER

    pltpu.sync_copy(genre_tab, gtab_v)

    for c in range(NCHUNKS):
        rb = base + c * CHUNK

        pltpu.sync_copy(title_idx.at[pl.ds(rb, CHUNK)], tidx_v)
        for j in range(N_GENRES):
            pltpu.sync_copy(genres_t.at[j, pl.ds(rb, CHUNK)], gidx_v.at[j])
        pltpu.sync_copy(text_flat.at[pl.ds(rb * TEXT_LEN, TOK_PER_CHUNK)],
                        xtok_v)
        xrow0 = wid * (NCHUNKS * NGATHER) + c * NGATHER
        pltpu.sync_copy(text_idx2.at[pl.ds(xrow0, NGATHER)], xidx_v)

        tcp = pltpu.async_copy(title_tab.at[tidx_v], trows_v, sem)
        xcps = []
        for j in range(NGATHER):
            xcps.append(pltpu.async_copy(
                text_tab.at[xidx_v.at[j]],
                xrows_v.at[pl.ds(j * IDX_W, IDX_W)], sem))
        tcp.wait()
        for cp in xcps:
            cp.wait()

        def row_body(b, carry):
            out_v[b, pl.ds(0, 16)] = trows_v[b, pl.ds(0, 16)]
            out_v[b, pl.ds(16, 16)] = trows_v[b, pl.ds(16, 16)]
            g0 = jnp.zeros((16,), jnp.float32)
            g1 = jnp.zeros((16,), jnp.float32)
            for j in range(N_GENRES):
                g = gidx_v[j, b]
                g0 = g0 + gtab_v[g, pl.ds(0, 16)]
                g1 = g1 + gtab_v[g, pl.ds(16, 16)]
            out_v[b, pl.ds(32, 16)] = g0 * 0.25
            out_v[b, pl.ds(48, 16)] = g1 * 0.25
            t0 = jnp.zeros((16,), jnp.float32)
            t1 = jnp.zeros((16,), jnp.float32)
            cnt = jnp.float32(0.0)
            tb = b * TEXT_LEN
            for t in range(TEXT_LEN):
                tok = xtok_v[tb + t]
                w = jnp.where(tok != 0, jnp.float32(1.0), jnp.float32(0.0))
                cnt = cnt + w
                t0 = t0 + xrows_v[tb + t, pl.ds(0, 16)] * w
                t1 = t1 + xrows_v[tb + t, pl.ds(16, 16)] * w
            inv = jnp.float32(1.0) / jnp.maximum(cnt, jnp.float32(1.0))
            out_v[b, pl.ds(64, 16)] = t0 * inv
            out_v[b, pl.ds(80, 16)] = t1 * inv
            return carry

        lax.fori_loop(0, CHUNK, row_body, None)

        pltpu.sync_copy(out_v, out.at[pl.ds(rb, CHUNK)])


@jax.jit
def _run(title_idx, genres_t, text_idx2, text_flat,
         title_tab, genre_tab, text_tab):
    mesh = plsc.VectorSubcoreMesh(core_axis_name="c", subcore_axis_name="s")
    fn = pl.kernel(
        _body,
        out_type=jax.ShapeDtypeStruct((B, 3 * EMB), jnp.float32),
        mesh=mesh,
        scratch_types=[
            pltpu.VMEM((GENRE_VOCAB, EMB), jnp.float32),
            pltpu.VMEM((CHUNK,), jnp.int32),
            pltpu.VMEM((CHUNK, EMB), jnp.float32),
            pltpu.VMEM((N_GENRES, CHUNK), jnp.int32),
            pltpu.VMEM((NGATHER, IDX_W), jnp.int32),
            pltpu.VMEM((TOK_PER_CHUNK,), jnp.int32),
            pltpu.VMEM((TOK_PER_CHUNK, EMB), jnp.float32),
            pltpu.VMEM((CHUNK, 3 * EMB), jnp.float32),
            pltpu.SemaphoreType.DMA,
        ],
    )
    return fn(title_idx, genres_t, text_idx2, text_flat,
              title_tab, genre_tab, text_tab)


def kernel(movie_title, movie_genres, movie_title_text,
           title_table, genre_table, text_table):
    title_idx = movie_title.astype(jnp.int32)
    genres_t = movie_genres.astype(jnp.int32).T
    text_flat = movie_title_text.astype(jnp.int32).reshape(-1)
    text_idx2 = text_flat.reshape(-1, IDX_W)
    return _run(title_idx, genres_t, text_idx2, text_flat,
                title_table, genre_table, text_table)


# speedup vs baseline: 4.0839x; 4.0839x over previous
import jax
import jax.numpy as jnp
from jax import lax
from jax.experimental import pallas as pl
from jax.experimental.pallas import tpu as pltpu
from jax.experimental.pallas import tpu_sc as plsc

B = 16384
EMB = 32
N_GENRES = 4
TEXT_LEN = 20
GENRE_VOCAB = 21
TEXT_VOCAB = 10000

NUM_WORKERS = 32
ROWS_PER_WORKER = B // NUM_WORKERS
CHUNK = 64
NGROUPS = CHUNK // 16
NCHUNKS = ROWS_PER_WORKER // CHUNK
TOK_PER_CHUNK = CHUNK * TEXT_LEN
IDX_W = 128
NGATHER = TOK_PER_CHUNK // IDX_W


def _body(title_idx, genres_f, text_tf, text_idxf,
          title_tab, genre_tab, text_tab, out,
          gtab_v, tidx_v, trows_v, gidx_v, ttok_v, xidx_v, xrows_v,
          out_v, sem):
    wid = lax.axis_index("s") * 2 + lax.axis_index("c")
    base = wid * ROWS_PER_WORKER

    pltpu.sync_copy(genre_tab, gtab_v)

    lane = jax.lax.iota(jnp.int32, 16)

    def chunk_body(c, chunk_carry):
        rb = base + c * CHUNK

        pltpu.sync_copy(title_idx.at[pl.ds(rb, CHUNK)], tidx_v)
        for j in range(N_GENRES):
            pltpu.sync_copy(genres_f.at[pl.ds(j * B + rb, CHUNK)],
                            gidx_v.at[j])
        for t in range(TEXT_LEN):
            pltpu.sync_copy(text_tf.at[pl.ds(t * B + rb, CHUNK)],
                            ttok_v.at[t])
        for j in range(NGATHER):
            pltpu.sync_copy(
                text_idxf.at[pl.ds(rb * TEXT_LEN + j * IDX_W, IDX_W)],
                xidx_v.at[j])

        tcp = pltpu.async_copy(title_tab.at[tidx_v], trows_v, sem)
        xcps = []
        for j in range(NGATHER):
            xcps.append(pltpu.async_copy(
                text_tab.at[xidx_v.at[j]],
                xrows_v.at[pl.ds(j * IDX_W, IDX_W)], sem))
        tcp.wait()
        for cp in xcps:
            cp.wait()

        def group_body(g, group_carry):
            b16 = lane + g * 16
            ob16 = b16 * (3 * EMB)
            trow16 = b16 * TEXT_LEN
            gj = [gidx_v[j, pl.ds(g * 16, 16)] for j in range(N_GENRES)]
            cnt = jnp.zeros((16,), jnp.float32)
            for t in range(TEXT_LEN):
                tok = ttok_v[t, pl.ds(g * 16, 16)]
                cnt = cnt + jnp.where(tok != 0, 1.0, 0.0).astype(jnp.float32)
            inv = jnp.float32(1.0) / jnp.maximum(cnt, jnp.float32(1.0))

            def dim_body(d, carry):
                dsplat = jnp.zeros((16,), jnp.int32) + d
                tv = plsc.load_gather(trows_v, [b16, dsplat])
                plsc.store_scatter(out_v, [ob16 + d], tv)
                ga = plsc.load_gather(gtab_v, [gj[0], dsplat])
                for j in range(1, N_GENRES):
                    ga = ga + plsc.load_gather(gtab_v, [gj[j], dsplat])
                plsc.store_scatter(out_v, [ob16 + (d + EMB)], ga * 0.25)
                xa = plsc.load_gather(xrows_v, [trow16, dsplat])
                for t in range(1, TEXT_LEN):
                    xa = xa + plsc.load_gather(xrows_v, [trow16 + t, dsplat])
                plsc.store_scatter(out_v, [ob16 + (d + 2 * EMB)], xa * inv)
                return carry

            lax.fori_loop(0, EMB, dim_body, None)
            return group_carry

        lax.fori_loop(0, NGROUPS, group_body, None)

        pltpu.sync_copy(out_v, out.at[pl.ds(rb * 3 * EMB, CHUNK * 3 * EMB)])
        return chunk_carry

    lax.fori_loop(0, NCHUNKS, chunk_body, None)


@jax.jit
def _run(title_idx, genres_f, text_tf, text_idxf,
         title_tab, genre_tab, text_tab):
    mesh = plsc.VectorSubcoreMesh(core_axis_name="c", subcore_axis_name="s")
    fn = pl.kernel(
        _body,
        out_type=jax.ShapeDtypeStruct((B * 3 * EMB,), jnp.float32),
        mesh=mesh,
        scratch_types=[
            pltpu.VMEM((GENRE_VOCAB, EMB), jnp.float32),
            pltpu.VMEM((CHUNK,), jnp.int32),
            pltpu.VMEM((CHUNK, EMB), jnp.float32),
            pltpu.VMEM((N_GENRES, CHUNK), jnp.int32),
            pltpu.VMEM((TEXT_LEN, CHUNK), jnp.int32),
            pltpu.VMEM((NGATHER, IDX_W), jnp.int32),
            pltpu.VMEM((TOK_PER_CHUNK, EMB), jnp.float32),
            pltpu.VMEM((CHUNK * 3 * EMB,), jnp.float32),
            pltpu.SemaphoreType.DMA,
        ],
        compiler_params=pltpu.CompilerParams(needs_layout_passes=False,
                                             use_tc_tiling_on_sc=False),
    )
    return fn(title_idx, genres_f, text_tf, text_idxf,
              title_tab, genre_tab, text_tab)


def kernel(movie_title, movie_genres, movie_title_text,
           title_table, genre_table, text_table):
    title_idx = movie_title.astype(jnp.int32)
    genres_f = movie_genres.astype(jnp.int32).T.reshape(-1)
    toks = movie_title_text.astype(jnp.int32)
    text_tf = toks.T.reshape(-1)
    text_idxf = jnp.where(toks == 0, TEXT_VOCAB, toks).reshape(-1)
    text_tab_ext = jnp.concatenate(
        [text_table, jnp.zeros((1, EMB), jnp.float32)], axis=0)
    flat = _run(title_idx, genres_f, text_tf, text_idxf,
                title_table, genre_table, text_tab_ext)
    return flat.reshape(B, 3 * EMB)


# speedup vs baseline: 5.0442x; 1.2351x over previous
import jax
import jax.numpy as jnp
from jax import lax
from jax.experimental import pallas as pl
from jax.experimental.pallas import tpu as pltpu
from jax.experimental.pallas import tpu_sc as plsc

B = 16384
EMB = 32
N_GENRES = 4
TEXT_LEN = 20
GENRE_VOCAB = 21
TEXT_VOCAB = 10000

NUM_WORKERS = 32
ROWS_PER_WORKER = B // NUM_WORKERS
CHUNK = 64
NGROUPS = CHUNK // 16
NCHUNKS = ROWS_PER_WORKER // CHUNK
TOK_PER_CHUNK = CHUNK * TEXT_LEN
IDX_W = 128
NGATHER = TOK_PER_CHUNK // IDX_W


def _body(title_idx, genres_bf, text_idxf,
          title_tab, genre_tab, text_tab, out,
          gtab_v, tidx_v, trows_v, gidx_v, xidx_v, xrows_v,
          out_v, sem, gsem):
    wid = lax.axis_index("s") * 2 + lax.axis_index("c")
    base = wid * ROWS_PER_WORKER

    pltpu.sync_copy(genre_tab, gtab_v)

    lane = jax.lax.iota(jnp.int32, 16)

    def chunk_body(c, chunk_carry):
        rb = base + c * CHUNK

        cps = [
            pltpu.async_copy(title_idx.at[pl.ds(rb, CHUNK)], tidx_v, sem),
            pltpu.async_copy(genres_bf.at[pl.ds(rb * N_GENRES,
                                                CHUNK * N_GENRES)],
                             gidx_v, sem),
        ]
        for j in range(NGATHER):
            cps.append(pltpu.async_copy(
                text_idxf.at[pl.ds(rb * TEXT_LEN + j * IDX_W, IDX_W)],
                xidx_v.at[j], sem))
        for cp in cps:
            cp.wait()

        gcps = [pltpu.async_copy(title_tab.at[tidx_v], trows_v, gsem)]
        for j in range(NGATHER):
            gcps.append(pltpu.async_copy(
                text_tab.at[xidx_v.at[j]],
                xrows_v.at[pl.ds(j * IDX_W, IDX_W)], gsem))
        for cp in gcps:
            cp.wait()

        def group_body(g, group_carry):
            b16 = lane + g * 16
            ob16 = b16 * (3 * EMB)
            trow16 = b16 * TEXT_LEN
            qb16 = b16 * N_GENRES
            gj = [plsc.load_gather(gidx_v, [qb16 + j])
                  for j in range(N_GENRES)]
            cnt = jnp.zeros((16,), jnp.float32)
            for t in range(TEXT_LEN):
                p = trow16 + t
                tok = plsc.load_gather(xidx_v, [p >> 7, p & 127])
                cnt = cnt + jnp.where(tok != TEXT_VOCAB, 1.0,
                                      0.0).astype(jnp.float32)
            inv = jnp.float32(1.0) / jnp.maximum(cnt, jnp.float32(1.0))

            def dim_body(d, carry):
                dsplat = jnp.zeros((16,), jnp.int32) + d
                tv = plsc.load_gather(trows_v, [b16, dsplat])
                plsc.store_scatter(out_v, [ob16 + d], tv)
                ga = plsc.load_gather(gtab_v, [gj[0], dsplat])
                for j in range(1, N_GENRES):
                    ga = ga + plsc.load_gather(gtab_v, [gj[j], dsplat])
                plsc.store_scatter(out_v, [ob16 + (d + EMB)], ga * 0.25)
                xa = plsc.load_gather(xrows_v, [trow16, dsplat])
                for t in range(1, TEXT_LEN):
                    xa = xa + plsc.load_gather(xrows_v, [trow16 + t, dsplat])
                plsc.store_scatter(out_v, [ob16 + (d + 2 * EMB)], xa * inv)
                return carry

            lax.fori_loop(0, EMB, dim_body, None)
            return group_carry

        lax.fori_loop(0, NGROUPS, group_body, None)

        pltpu.sync_copy(out_v, out.at[pl.ds(rb * 3 * EMB, CHUNK * 3 * EMB)])
        return chunk_carry

    lax.fori_loop(0, NCHUNKS, chunk_body, None)


@jax.jit
def _run(title_idx, genres_bf, text_idxf, title_tab, genre_tab, text_tab):
    mesh = plsc.VectorSubcoreMesh(core_axis_name="c", subcore_axis_name="s")
    fn = pl.kernel(
        _body,
        out_type=jax.ShapeDtypeStruct((B * 3 * EMB,), jnp.float32),
        mesh=mesh,
        scratch_types=[
            pltpu.VMEM((GENRE_VOCAB, EMB), jnp.float32),
            pltpu.VMEM((CHUNK,), jnp.int32),
            pltpu.VMEM((CHUNK, EMB), jnp.float32),
            pltpu.VMEM((CHUNK * N_GENRES,), jnp.int32),
            pltpu.VMEM((NGATHER, IDX_W), jnp.int32),
            pltpu.VMEM((TOK_PER_CHUNK, EMB), jnp.float32),
            pltpu.VMEM((CHUNK * 3 * EMB,), jnp.float32),
            pltpu.SemaphoreType.DMA,
            pltpu.SemaphoreType.DMA,
        ],
        compiler_params=pltpu.CompilerParams(needs_layout_passes=False,
                                             use_tc_tiling_on_sc=False),
    )
    return fn(title_idx, genres_bf, text_idxf, title_tab, genre_tab, text_tab)


def kernel(movie_title, movie_genres, movie_title_text,
           title_table, genre_table, text_table):
    title_idx = movie_title.astype(jnp.int32)
    genres_bf = movie_genres.astype(jnp.int32).reshape(-1)
    toks = movie_title_text.astype(jnp.int32)
    text_idxf = jnp.where(toks == 0, TEXT_VOCAB, toks).reshape(-1)
    text_tab_ext = jnp.concatenate(
        [text_table, jnp.zeros((1, EMB), jnp.float32)], axis=0)
    flat = _run(title_idx, genres_bf, text_idxf,
                title_table, genre_table, text_tab_ext)
    return flat.reshape(B, 3 * EMB)


# speedup vs baseline: 5.2925x; 1.0492x over previous
import jax
import jax.numpy as jnp
from jax import lax
from jax.experimental import pallas as pl
from jax.experimental.pallas import tpu as pltpu
from jax.experimental.pallas import tpu_sc as plsc

B = 16384
EMB = 32
N_GENRES = 4
TEXT_LEN = 20
GENRE_VOCAB = 21
TEXT_VOCAB = 10000

NUM_WORKERS = 32
ROWS_PER_WORKER = B // NUM_WORKERS
CHUNK = 64
NGROUPS = CHUNK // 16
NCHUNKS = ROWS_PER_WORKER // CHUNK
TOK_PER_CHUNK = CHUNK * TEXT_LEN
IDX_W = 128
NGATHER = TOK_PER_CHUNK // IDX_W


def _tree_sum(vals):
    while len(vals) > 1:
        nxt = [vals[i] + vals[i + 1] for i in range(0, len(vals) - 1, 2)]
        if len(vals) % 2:
            nxt.append(vals[-1])
        vals = nxt
    return vals[0]


def _body(title_idx, genres_bf, text_idxf,
          title_tab, genre_tab, text_tab, out,
          gtab_v, tidx_v, trows_v, gidx_v, xidx_v, xrows_v,
          out_v, sem_s, sem_g, sem_o):
    wid = lax.axis_index("s") * 2 + lax.axis_index("c")
    base = wid * ROWS_PER_WORKER

    pltpu.sync_copy(genre_tab, gtab_v)

    lane = jax.lax.iota(jnp.int32, 16)

    def fire_stage(c):
        p = c % 2
        rb = base + c * CHUNK
        cps = [
            pltpu.async_copy(title_idx.at[pl.ds(rb, CHUNK)],
                             tidx_v[p], sem_s[p]),
            pltpu.async_copy(genres_bf.at[pl.ds(rb * N_GENRES,
                                                CHUNK * N_GENRES)],
                             gidx_v[p], sem_s[p]),
        ]
        for j in range(NGATHER):
            cps.append(pltpu.async_copy(
                text_idxf.at[pl.ds(rb * TEXT_LEN + j * IDX_W, IDX_W)],
                xidx_v[p].at[j], sem_s[p]))
        return cps

    def fire_gathers(c):
        p = c % 2
        gcps = [pltpu.async_copy(title_tab.at[tidx_v[p]],
                                 trows_v[p], sem_g[p])]
        for j in range(NGATHER):
            gcps.append(pltpu.async_copy(
                text_tab.at[xidx_v[p].at[j]],
                xrows_v[p].at[pl.ds(j * IDX_W, IDX_W)], sem_g[p]))
        return gcps

    def compute(c):
        p = c % 2

        def group_body(g, group_carry):
            b16 = lane + g * 16
            ob16 = b16 * (3 * EMB)
            trow16 = b16 * TEXT_LEN
            qb16 = b16 * N_GENRES
            gj = [plsc.load_gather(gidx_v[p], [qb16 + j])
                  for j in range(N_GENRES)]
            ws = []
            for t in range(TEXT_LEN):
                pos = trow16 + t
                tok = plsc.load_gather(xidx_v[p], [pos >> 7, pos & 127])
                ws.append(jnp.where(tok != TEXT_VOCAB, 1.0,
                                    0.0).astype(jnp.float32))
            inv = jnp.float32(1.0) / jnp.maximum(_tree_sum(ws),
                                                 jnp.float32(1.0))

            def dim_body(d, carry):
                dsplat = jnp.zeros((16,), jnp.int32) + d
                tv = plsc.load_gather(trows_v[p], [b16, dsplat])
                plsc.store_scatter(out_v[p], [ob16 + d], tv)
                ga = _tree_sum([plsc.load_gather(gtab_v, [gj[j], dsplat])
                                for j in range(N_GENRES)])
                plsc.store_scatter(out_v[p], [ob16 + (d + EMB)],
                                   ga * 0.25)
                xa = _tree_sum(
                    [plsc.load_gather(xrows_v[p], [trow16 + t, dsplat])
                     for t in range(TEXT_LEN)])
                plsc.store_scatter(out_v[p], [ob16 + (d + 2 * EMB)],
                                   xa * inv)
                return carry

            lax.fori_loop(0, EMB, dim_body, None, unroll=2)
            return group_carry

        lax.fori_loop(0, NGROUPS, group_body, None)

    def fire_out(c):
        p = c % 2
        rb = base + c * CHUNK
        return [pltpu.async_copy(
            out_v[p], out.at[pl.ds(rb * 3 * EMB, CHUNK * 3 * EMB)],
            sem_o[p])]

    stage_cps = {0: fire_stage(0)}
    for cp in stage_cps[0]:
        cp.wait()
    gather_cps = {0: fire_gathers(0)}
    stage_cps[1] = fire_stage(1)
    out_cps = {}
    for c in range(NCHUNKS):
        if c + 1 < NCHUNKS:
            for cp in stage_cps[c + 1]:
                cp.wait()
            gather_cps[c + 1] = fire_gathers(c + 1)
        for cp in gather_cps[c]:
            cp.wait()
        if c >= 2:
            for cp in out_cps[c - 2]:
                cp.wait()
        compute(c)
        out_cps[c] = fire_out(c)
        if c + 2 < NCHUNKS:
            stage_cps[c + 2] = fire_stage(c + 2)
    for cp in out_cps[NCHUNKS - 2] + out_cps[NCHUNKS - 1]:
        cp.wait()


@jax.jit
def _run(title_idx, genres_bf, text_idxf, title_tab, genre_tab, text_tab):
    mesh = plsc.VectorSubcoreMesh(core_axis_name="c", subcore_axis_name="s")
    fn = pl.kernel(
        _body,
        out_type=jax.ShapeDtypeStruct((B * 3 * EMB,), jnp.float32),
        mesh=mesh,
        scratch_types=[
            pltpu.VMEM((GENRE_VOCAB, EMB), jnp.float32),
            [pltpu.VMEM((CHUNK,), jnp.int32)] * 2,
            [pltpu.VMEM((CHUNK, EMB), jnp.float32)] * 2,
            [pltpu.VMEM((CHUNK * N_GENRES,), jnp.int32)] * 2,
            [pltpu.VMEM((NGATHER, IDX_W), jnp.int32)] * 2,
            [pltpu.VMEM((TOK_PER_CHUNK, EMB), jnp.float32)] * 2,
            [pltpu.VMEM((CHUNK * 3 * EMB,), jnp.float32)] * 2,
            [pltpu.SemaphoreType.DMA] * 2,
            [pltpu.SemaphoreType.DMA] * 2,
            [pltpu.SemaphoreType.DMA] * 2,
        ],
        compiler_params=pltpu.CompilerParams(needs_layout_passes=False,
                                             use_tc_tiling_on_sc=False),
    )
    return fn(title_idx, genres_bf, text_idxf, title_tab, genre_tab, text_tab)


def kernel(movie_title, movie_genres, movie_title_text,
           title_table, genre_table, text_table):
    title_idx = movie_title.astype(jnp.int32)
    genres_bf = movie_genres.astype(jnp.int32).reshape(-1)
    toks = movie_title_text.astype(jnp.int32)
    text_idxf = jnp.where(toks == 0, TEXT_VOCAB, toks).reshape(-1)
    text_tab_ext = jnp.concatenate(
        [text_table, jnp.zeros((1, EMB), jnp.float32)], axis=0)
    flat = _run(title_idx, genres_bf, text_idxf,
                title_table, genre_table, text_tab_ext)
    return flat.reshape(B, 3 * EMB)


# speedup vs baseline: 7.4413x; 1.4060x over previous
import jax
import jax.numpy as jnp
from jax import lax
from jax.experimental import pallas as pl
from jax.experimental.pallas import tpu as pltpu
from jax.experimental.pallas import tpu_sc as plsc

B = 16384
EMB = 32
N_GENRES = 4
TEXT_LEN = 20
TEXT_VOCAB = 10000

NUM_WORKERS = 32
ROWS_PER_WORKER = B // NUM_WORKERS
CHUNK = 64
NGROUPS = CHUNK // 16
NCHUNKS = ROWS_PER_WORKER // CHUNK
TOK_PER_CHUNK = CHUNK * TEXT_LEN
IDX_W = 128
NGATHER = TOK_PER_CHUNK // IDX_W
NGG = CHUNK * N_GENRES // IDX_W
IVW = 17


def _tree_sum(vals):
    while len(vals) > 1:
        nxt = [vals[i] + vals[i + 1] for i in range(0, len(vals) - 1, 2)]
        if len(vals) % 2:
            nxt.append(vals[-1])
        vals = nxt
    return vals[0]


def _body(title_idx, genres_bf, text_idxf, ttok_blk,
          title_tab, genre_tab, text_tab, out,
          tidx_v, gidx_v, xidx_v, ttok_v, trows_v, grows_v, xrows_v,
          invb_v, out_v, sem_s, sem_g, sem_o):
    wid = lax.axis_index("s") * 2 + lax.axis_index("c")
    base = wid * ROWS_PER_WORKER

    lane = jax.lax.iota(jnp.int32, 16)

    def fire_stage(c):
        p = c % 2
        rb = base + c * CHUNK
        cps = [
            pltpu.async_copy(title_idx.at[pl.ds(rb, CHUNK)],
                             tidx_v[p], sem_s[p]),
        ]
        for j in range(NGG):
            cps.append(pltpu.async_copy(
                genres_bf.at[pl.ds(rb * N_GENRES + j * IDX_W, IDX_W)],
                gidx_v[p].at[j], sem_s[p]))
        for j in range(NGATHER):
            cps.append(pltpu.async_copy(
                text_idxf.at[pl.ds(rb * TEXT_LEN + j * IDX_W, IDX_W)],
                xidx_v[p].at[j], sem_s[p]))
        cps.append(pltpu.async_copy(
            ttok_blk.at[pl.ds(rb * TEXT_LEN, TOK_PER_CHUNK)],
            ttok_v[p], sem_s[p]))
        return cps

    def fire_gathers(c):
        p = c % 2
        gcps = [pltpu.async_copy(title_tab.at[tidx_v[p]],
                                 trows_v[p], sem_g[p])]
        for j in range(NGG):
            gcps.append(pltpu.async_copy(
                genre_tab.at[gidx_v[p].at[j]],
                grows_v[p].at[pl.ds(j * IDX_W, IDX_W)], sem_g[p]))
        for j in range(NGATHER):
            gcps.append(pltpu.async_copy(
                text_tab.at[xidx_v[p].at[j]],
                xrows_v[p].at[pl.ds(j * IDX_W, IDX_W)], sem_g[p]))
        return gcps

    def compute(c):
        p = c % 2

        def group_body(g, group_carry):
            ws = []
            for t in range(TEXT_LEN):
                tok = ttok_v[p][pl.ds(g * 16 + t * CHUNK, 16)]
                ws.append(jnp.where(tok != TEXT_VOCAB, 1.0,
                                    0.0).astype(jnp.float32))
            inv = jnp.float32(1.0) / jnp.maximum(_tree_sum(ws),
                                                 jnp.float32(1.0))
            ob = (lane + g * 16) * IVW
            for k in range(16):
                plsc.store_scatter(invb_v[p], [ob + k], inv)
            return group_carry

        lax.fori_loop(0, NGROUPS, group_body, None)

        def row_body(b, row_carry):
            iv = invb_v[p][pl.ds(b * IVW, 16)]
            for h in (0, 16):
                tv = trows_v[p][b, pl.ds(h, 16)]
                out_v[p][pl.ds(b * 3 * EMB + h, 16)] = tv
                ga = _tree_sum([grows_v[p][b * N_GENRES + j, pl.ds(h, 16)]
                                for j in range(N_GENRES)])
                out_v[p][pl.ds(b * 3 * EMB + EMB + h, 16)] = ga * 0.25
                xa = _tree_sum([xrows_v[p][b * TEXT_LEN + t, pl.ds(h, 16)]
                                for t in range(TEXT_LEN)])
                out_v[p][pl.ds(b * 3 * EMB + 2 * EMB + h, 16)] = xa * iv
            return row_carry

        lax.fori_loop(0, CHUNK, row_body, None, unroll=2)

    def fire_out(c):
        p = c % 2
        rb = base + c * CHUNK
        return [pltpu.async_copy(
            out_v[p], out.at[pl.ds(rb * 3 * EMB, CHUNK * 3 * EMB)],
            sem_o[p])]

    stage_cps = {0: fire_stage(0)}
    for cp in stage_cps[0]:
        cp.wait()
    gather_cps = {0: fire_gathers(0)}
    stage_cps[1] = fire_stage(1)
    out_cps = {}
    for c in range(NCHUNKS):
        if c + 1 < NCHUNKS:
            for cp in stage_cps[c + 1]:
                cp.wait()
            gather_cps[c + 1] = fire_gathers(c + 1)
        for cp in gather_cps[c]:
            cp.wait()
        if c >= 2:
            for cp in out_cps[c - 2]:
                cp.wait()
        compute(c)
        out_cps[c] = fire_out(c)
        if c + 2 < NCHUNKS:
            stage_cps[c + 2] = fire_stage(c + 2)
    for cp in out_cps[NCHUNKS - 2] + out_cps[NCHUNKS - 1]:
        cp.wait()


@jax.jit
def _run(title_idx, genres_bf, text_idxf, ttok_blk,
         title_tab, genre_tab, text_tab):
    mesh = plsc.VectorSubcoreMesh(core_axis_name="c", subcore_axis_name="s")
    fn = pl.kernel(
        _body,
        out_type=jax.ShapeDtypeStruct((B * 3 * EMB,), jnp.float32),
        mesh=mesh,
        scratch_types=[
            [pltpu.VMEM((CHUNK,), jnp.int32)] * 2,
            [pltpu.VMEM((NGG, IDX_W), jnp.int32)] * 2,
            [pltpu.VMEM((NGATHER, IDX_W), jnp.int32)] * 2,
            [pltpu.VMEM((TOK_PER_CHUNK,), jnp.int32)] * 2,
            [pltpu.VMEM((CHUNK, EMB), jnp.float32)] * 2,
            [pltpu.VMEM((CHUNK * N_GENRES, EMB), jnp.float32)] * 2,
            [pltpu.VMEM((TOK_PER_CHUNK, EMB), jnp.float32)] * 2,
            [pltpu.VMEM((CHUNK * IVW,), jnp.float32)] * 2,
            [pltpu.VMEM((CHUNK * 3 * EMB,), jnp.float32)] * 2,
            [pltpu.SemaphoreType.DMA] * 2,
            [pltpu.SemaphoreType.DMA] * 2,
            [pltpu.SemaphoreType.DMA] * 2,
        ],
        compiler_params=pltpu.CompilerParams(needs_layout_passes=False,
                                             use_tc_tiling_on_sc=False),
    )
    return fn(title_idx, genres_bf, text_idxf, ttok_blk,
              title_tab, genre_tab, text_tab)


def kernel(movie_title, movie_genres, movie_title_text,
           title_table, genre_table, text_table):
    title_idx = movie_title.astype(jnp.int32)
    genres_bf = movie_genres.astype(jnp.int32).reshape(-1)
    toks = movie_title_text.astype(jnp.int32)
    remap = jnp.where(toks == 0, TEXT_VOCAB, toks)
    text_idxf = remap.reshape(-1)
    ttok_blk = (remap.reshape(B // CHUNK, CHUNK, TEXT_LEN)
                .transpose(0, 2, 1).reshape(-1))
    text_tab_ext = jnp.concatenate(
        [text_table, jnp.zeros((1, EMB), jnp.float32)], axis=0)
    flat = _run(title_idx, genres_bf, text_idxf, ttok_blk,
                title_table, genre_table, text_tab_ext)
    return flat.reshape(B, 3 * EMB)


# speedup vs baseline: 7.5590x; 1.0158x over previous
import jax
import jax.numpy as jnp
from jax import lax
from jax.experimental import pallas as pl
from jax.experimental.pallas import tpu as pltpu
from jax.experimental.pallas import tpu_sc as plsc

B = 16384
EMB = 32
N_GENRES = 4
TEXT_LEN = 20

NUM_WORKERS = 32
ROWS_PER_WORKER = B // NUM_WORKERS
CHUNK = 64
NGROUPS = CHUNK // 16
NCHUNKS = ROWS_PER_WORKER // CHUNK
TOK_PER_CHUNK = CHUNK * TEXT_LEN
IDX_W = 128
NGATHER = TOK_PER_CHUNK // IDX_W
NGG = CHUNK * N_GENRES // IDX_W
IVW = 17


def _tree_sum(vals):
    while len(vals) > 1:
        nxt = [vals[i] + vals[i + 1] for i in range(0, len(vals) - 1, 2)]
        if len(vals) % 2:
            nxt.append(vals[-1])
        vals = nxt
    return vals[0]


def _body(title_idx, genres_bf, text_idxf,
          title_tab, genre_tab, text_tab, out,
          tidx_v, gidx_v, xidx_v, trows_v, grows_v, xrows_v,
          invb_v, zb_v, r0_v, out_v, sem_s, sem_g, sem_o):
    wid = lax.axis_index("s") * 2 + lax.axis_index("c")
    base = wid * ROWS_PER_WORKER

    lane = jax.lax.iota(jnp.int32, 16)

    pltpu.sync_copy(text_tab.at[pl.ds(0, 8)], r0_v)
    r0 = [r0_v[0, pl.ds(h, 16)] for h in (0, 16)]

    def fire_stage(c):
        p = c % 2
        rb = base + c * CHUNK
        cps = [
            pltpu.async_copy(title_idx.at[pl.ds(rb, CHUNK)],
                             tidx_v[p], sem_s[p]),
        ]
        for j in range(NGG):
            cps.append(pltpu.async_copy(
                genres_bf.at[pl.ds(rb * N_GENRES + j * IDX_W, IDX_W)],
                gidx_v[p].at[j], sem_s[p]))
        for j in range(NGATHER):
            cps.append(pltpu.async_copy(
                text_idxf.at[pl.ds(rb * TEXT_LEN + j * IDX_W, IDX_W)],
                xidx_v[p].at[j], sem_s[p]))
        return cps

    def fire_gathers(c):
        p = c % 2
        gcps = [pltpu.async_copy(title_tab.at[tidx_v[p]],
                                 trows_v[p], sem_g[p])]
        for j in range(NGG):
            gcps.append(pltpu.async_copy(
                genre_tab.at[gidx_v[p].at[j]],
                grows_v[p].at[pl.ds(j * IDX_W, IDX_W)], sem_g[p]))
        for j in range(NGATHER):
            gcps.append(pltpu.async_copy(
                text_tab.at[xidx_v[p].at[j]],
                xrows_v[p].at[pl.ds(j * IDX_W, IDX_W)], sem_g[p]))
        return gcps

    def compute(c):
        p = c % 2

        def group_body(g, group_carry):
            trow16 = (lane + g * 16) * TEXT_LEN
            ws = []
            for t in range(TEXT_LEN):
                pos = trow16 + t
                tok = plsc.load_gather(xidx_v[p], [pos >> 7, pos & 127])
                ws.append(jnp.where(tok != 0, 1.0, 0.0).astype(jnp.float32))
            cnt = _tree_sum(ws)
            inv = jnp.float32(1.0) / jnp.maximum(cnt, jnp.float32(1.0))
            nz = jnp.float32(TEXT_LEN) - cnt
            ob = (lane + g * 16) * IVW
            for k in range(16):
                plsc.store_scatter(invb_v[p], [ob + k], inv)
                plsc.store_scatter(zb_v[p], [ob + k], nz)
            return group_carry

        lax.fori_loop(0, NGROUPS, group_body, None)

        def row_body(b, row_carry):
            iv = invb_v[p][pl.ds(b * IVW, 16)]
            nz = zb_v[p][pl.ds(b * IVW, 16)]
            for hi, h in enumerate((0, 16)):
                tv = trows_v[p][b, pl.ds(h, 16)]
                out_v_p = out_v[p]
                out_v_p[pl.ds(b * 3 * EMB + h, 16)] = tv
                ga = _tree_sum([grows_v[p][b * N_GENRES + j, pl.ds(h, 16)]
                                for j in range(N_GENRES)])
                out_v_p[pl.ds(b * 3 * EMB + EMB + h, 16)] = ga * 0.25
                xa = _tree_sum([xrows_v[p][b * TEXT_LEN + t, pl.ds(h, 16)]
                                for t in range(TEXT_LEN)])
                out_v_p[pl.ds(b * 3 * EMB + 2 * EMB + h, 16)] = (
                    (xa - nz * r0[hi]) * iv)
            return row_carry

        lax.fori_loop(0, CHUNK, row_body, None, unroll=2)

    def fire_out(c):
        p = c % 2
        rb = base + c * CHUNK
        return [pltpu.async_copy(
            out_v[p], out.at[pl.ds(rb * 3 * EMB, CHUNK * 3 * EMB)],
            sem_o[p])]

    stage_cps = {0: fire_stage(0)}
    for cp in stage_cps[0]:
        cp.wait()
    gather_cps = {0: fire_gathers(0)}
    stage_cps[1] = fire_stage(1)
    out_cps = {}
    for c in range(NCHUNKS):
        if c + 1 < NCHUNKS:
            for cp in stage_cps[c + 1]:
                cp.wait()
            gather_cps[c + 1] = fire_gathers(c + 1)
        for cp in gather_cps[c]:
            cp.wait()
        if c >= 2:
            for cp in out_cps[c - 2]:
                cp.wait()
        compute(c)
        out_cps[c] = fire_out(c)
        if c + 2 < NCHUNKS:
            stage_cps[c + 2] = fire_stage(c + 2)
    for cp in out_cps[NCHUNKS - 2] + out_cps[NCHUNKS - 1]:
        cp.wait()


@jax.jit
def _run(title_idx, genres_bf, text_idxf, title_tab, genre_tab, text_tab):
    mesh = plsc.VectorSubcoreMesh(core_axis_name="c", subcore_axis_name="s")
    fn = pl.kernel(
        _body,
        out_type=jax.ShapeDtypeStruct((B * 3 * EMB,), jnp.float32),
        mesh=mesh,
        scratch_types=[
            [pltpu.VMEM((CHUNK,), jnp.int32)] * 2,
            [pltpu.VMEM((NGG, IDX_W), jnp.int32)] * 2,
            [pltpu.VMEM((NGATHER, IDX_W), jnp.int32)] * 2,
            [pltpu.VMEM((CHUNK, EMB), jnp.float32)] * 2,
            [pltpu.VMEM((CHUNK * N_GENRES, EMB), jnp.float32)] * 2,
            [pltpu.VMEM((TOK_PER_CHUNK, EMB), jnp.float32)] * 2,
            [pltpu.VMEM((CHUNK * IVW,), jnp.float32)] * 2,
            [pltpu.VMEM((CHUNK * IVW,), jnp.float32)] * 2,
            pltpu.VMEM((8, EMB), jnp.float32),
            [pltpu.VMEM((CHUNK * 3 * EMB,), jnp.float32)] * 2,
            [pltpu.SemaphoreType.DMA] * 2,
            [pltpu.SemaphoreType.DMA] * 2,
            [pltpu.SemaphoreType.DMA] * 2,
        ],
        compiler_params=pltpu.CompilerParams(needs_layout_passes=False,
                                             use_tc_tiling_on_sc=False),
    )
    return fn(title_idx, genres_bf, text_idxf, title_tab, genre_tab, text_tab)


def kernel(movie_title, movie_genres, movie_title_text,
           title_table, genre_table, text_table):
    title_idx = movie_title.astype(jnp.int32)
    genres_bf = movie_genres.astype(jnp.int32).reshape(-1)
    text_idxf = movie_title_text.astype(jnp.int32).reshape(-1)
    flat = _run(title_idx, genres_bf, text_idxf,
                title_table, genre_table, text_table)
    return flat.reshape(B, 3 * EMB)
